# baseline (speedup 1.0000x reference)
import functools

import jax
import jax.numpy as jnp
from jax import lax
from jax.experimental import pallas as pl
from jax.experimental.pallas import tpu as pltpu
from jax.experimental.pallas import tpu_sc as plsc

F32 = jnp.float32
I32 = jnp.int32

HEADS = 8
HID = 128
NCLS = 40
NCLS_P = 48
IN_CH = 128
N = 10000
E = 320000

N_PAD = 10240
NB = N_PAD // 256
E1 = E + N
NW = 32
C = 96
KC = -(-E1 // NW)
KC = ((KC + C - 1) // C) * C
K = KC // C
E_PAD = NW * KC
NPS = N_PAD // 16
_PIECES = [(s, min(C, NPS - s)) for s in range(0, NPS, C)]


def _t0_body(x_ref, o_ref):
    i = pl.program_id(0)
    xb = x_ref[...]
    bs = jnp.sum(xb, axis=0, keepdims=True)
    bss = jnp.sum(xb * xb, axis=0, keepdims=True)

    @pl.when(i == 0)
    def _():
        o_ref[...] = jnp.zeros_like(o_ref)

    o_ref[0:1, :] += bs
    o_ref[1:2, :] += bss

    @pl.when(i == pl.num_programs(0) - 1)
    def _():
        m = o_ref[0:1, :] / N
        v = o_ref[1:2, :] / N - m * m
        o_ref[0:1, :] = m
        o_ref[1:2, :] = lax.rsqrt(v + 1e-5)


def _bn_stats(x):
    return pl.pallas_call(
        _t0_body,
        grid=(10,),
        in_specs=[pl.BlockSpec((1000, IN_CH), lambda i: (i, 0))],
        out_specs=pl.BlockSpec((8, IN_CH), lambda i: (0, 0)),
        out_shape=jax.ShapeDtypeStruct((8, IN_CH), F32),
    )(x)


def _row(ref, r):
    rows = lax.broadcasted_iota(I32, ref.shape, 0)
    return jnp.sum(jnp.where(rows == r, ref[...], 0.0), axis=0, keepdims=True)


def _t1_body(x_ref, st_ref, g_ref, b_ref, w_ref, as_ref, ad_ref,
             h_ref, acat_ref):
    h = pl.program_id(1)
    xn = (x_ref[...] - st_ref[0:1, :]) * st_ref[1:2, :] * g_ref[...] + b_ref[...]
    hb = jnp.dot(xn, w_ref[...], preferred_element_type=F32)
    h_ref[0] = hb
    asv = jnp.sum(hb * _row(as_ref, h), axis=1)
    adv = jnp.sum(hb * _row(ad_ref, h), axis=1)
    lane = lax.broadcasted_iota(I32, (256, 128), 1)

    @pl.when(h == 0)
    def _():
        acat_ref[...] = jnp.zeros_like(acat_ref)

    acat_ref[...] = jnp.where(lane == h, asv[:, None], acat_ref[...])
    acat_ref[...] = jnp.where(lane == 8 + h, adv[:, None], acat_ref[...])


def _layer1_dense(x_p, stats, gamma, beta, W1, att_src1, att_dst1):
    return pl.pallas_call(
        _t1_body,
        grid=(NB, HEADS),
        in_specs=[
            pl.BlockSpec((256, IN_CH), lambda n, h: (n, 0)),
            pl.BlockSpec((8, IN_CH), lambda n, h: (0, 0)),
            pl.BlockSpec((1, IN_CH), lambda n, h: (0, 0)),
            pl.BlockSpec((1, IN_CH), lambda n, h: (0, 0)),
            pl.BlockSpec((IN_CH, HID), lambda n, h: (0, h)),
            pl.BlockSpec((HEADS, HID), lambda n, h: (0, 0)),
            pl.BlockSpec((HEADS, HID), lambda n, h: (0, 0)),
        ],
        out_specs=[
            pl.BlockSpec((1, 256, HID), lambda n, h: (h, n, 0)),
            pl.BlockSpec((256, 128), lambda n, h: (n, 0)),
        ],
        out_shape=[
            jax.ShapeDtypeStruct((HEADS, N_PAD, HID), F32),
            jax.ShapeDtypeStruct((N_PAD, 128), F32),
        ],
    )(x_p, stats, gamma, beta, W1, att_src1, att_dst1)


def _cvec_body(acat_ref, o_ref):
    m = jnp.max(acat_ref[...], axis=0, keepdims=True)
    rolled = jnp.concatenate([m[:, 8:], m[:, :8]], axis=1)
    cs = m + rolled
    o_ref[...] = jnp.maximum(cs, 0.2 * cs)


def _cvec(acat):
    return pl.pallas_call(
        _cvec_body,
        out_shape=jax.ShapeDtypeStruct((1, 128), F32),
    )(acat)


def _t2_body(o1_ref, den_ref, b1_ref, w2_ref, ats_ref, atd_ref,
             h2_ref, acat_ref):
    ho2 = pl.program_id(1)
    hi = pl.program_id(2)
    lane16 = lax.broadcasted_iota(I32, (256, 16), 1)
    lane = lax.broadcasted_iota(I32, (256, 128), 1)
    d = den_ref[0] + den_ref[1]
    invcol = jnp.sum(jnp.where(lane16 == hi, 1.0 / (d + 1e-16), 0.0),
                     axis=1, keepdims=True)
    z = o1_ref[0] * invcol + _row(b1_ref, hi)
    z = jnp.where(z > 0, z, jnp.exp(jnp.minimum(z, 0.0)) - 1.0)
    contrib = jnp.dot(z, w2_ref[0], preferred_element_type=F32)

    @pl.when(hi == 0)
    def _():
        h2_ref[0] = contrib

    @pl.when(hi > 0)
    def _():
        h2_ref[0] = h2_ref[0] + contrib

    @pl.when(jnp.logical_and(ho2 == 0, hi == 0))
    def _():
        acat_ref[...] = jnp.zeros_like(acat_ref)

    @pl.when(hi == HEADS - 1)
    def _():
        hf = h2_ref[0]
        ats = _row(ats_ref, ho2)
        atd = _row(atd_ref, ho2)
        for hh in range(2):
            sl = slice(hh * NCLS_P, (hh + 1) * NCLS_P)
            asv = jnp.sum(hf[:, sl] * ats[0:1, sl], axis=1)
            adv = jnp.sum(hf[:, sl] * atd[0:1, sl], axis=1)
            hg = ho2 * 2 + hh
            acat_ref[...] = jnp.where(lane == hg, asv[:, None], acat_ref[...])
            acat_ref[...] = jnp.where(lane == 8 + hg, adv[:, None],
                                      acat_ref[...])


def _layer2_dense(out1, den1, b1r, W2q, ats2q, atd2q):
    return pl.pallas_call(
        _t2_body,
        grid=(NB, 4, HEADS),
        in_specs=[
            pl.BlockSpec((1, 256, HID), lambda n, o, i: (i, n, 0)),
            pl.BlockSpec((2, 256, 16), lambda n, o, i: (0, n, 0)),
            pl.BlockSpec((HEADS, HID), lambda n, o, i: (0, 0)),
            pl.BlockSpec((1, HID, 128), lambda n, o, i: (o, i, 0)),
            pl.BlockSpec((4, 2 * NCLS_P), lambda n, o, i: (0, 0)),
            pl.BlockSpec((4, 2 * NCLS_P), lambda n, o, i: (0, 0)),
        ],
        out_specs=[
            pl.BlockSpec((1, 256, 128), lambda n, o, i: (o, n, 0)),
            pl.BlockSpec((256, 128), lambda n, o, i: (n, 0)),
        ],
        out_shape=[
            jax.ShapeDtypeStruct((4, N_PAD, 128), F32),
            jax.ShapeDtypeStruct((N_PAD, 128), F32),
        ],
    )(out1, den1, b1r, W2q, ats2q, atd2q)


def _t3_body(o2_ref, den_ref, b2_ref, fin_ref):
    lane = lax.broadcasted_iota(I32, (256, 16), 1)
    inv = 1.0 / (den_ref[0] + den_ref[1] + 1e-16)
    for q in range(4):
        for hh in range(2):
            hg = q * 2 + hh
            col = jnp.sum(jnp.where(lane == hg, inv, 0.0),
                          axis=1, keepdims=True)
            sl_in = slice(hh * NCLS_P, (hh + 1) * NCLS_P)
            sl_out = slice(hg * NCLS_P, (hg + 1) * NCLS_P)
            fin_ref[:, sl_out] = (o2_ref[q, :, sl_in] * col
                                  + b2_ref[0:1, sl_out])


def _finalize(out2, den2, b2p):
    return pl.pallas_call(
        _t3_body,
        grid=(NB,),
        in_specs=[
            pl.BlockSpec((4, 256, 128), lambda n: (0, n, 0)),
            pl.BlockSpec((2, 256, 16), lambda n: (0, n, 0)),
            pl.BlockSpec((1, HEADS * NCLS_P), lambda n: (0, 0)),
        ],
        out_specs=pl.BlockSpec((256, HEADS * NCLS_P), lambda n: (n, 0)),
        out_shape=jax.ShapeDtypeStruct((N_PAD, HEADS * NCLS_P), F32),
    )(out2, den2, b2p)


_MESH = plsc.VectorSubcoreMesh(core_axis_name="c", subcore_axis_name="s")


def _lane_bcast(vec, idx16x1):
    return lax.gather(
        vec, idx16x1,
        lax.GatherDimensionNumbers(offset_dims=(), collapsed_slice_dims=(0,),
                                   start_index_map=(0,)),
        slice_sizes=(1,),
        mode=lax.GatherScatterMode.PROMISE_IN_BOUNDS)


def _edge_weights_kernel(src_hbm, dst_hbm, acat_hbm, c_hbm,
                         w_hbm, den_hbm,
                         sidx, didx, rows_s, rows_d, wbuf, cbuf,
                         den_sh, sem):
    cid = lax.axis_index("c")
    sid = lax.axis_index("s")
    wid = sid * 2 + cid
    base = wid * KC
    pltpu.sync_copy(c_hbm, cbuf)

    def zb(i, _):
        for r in range(8):
            rows_s[i, pl.ds(r * 16, 16)] = jnp.zeros((16,), F32)
        return 0
    lax.fori_loop(0, C, zb, 0)
    for (s0, nr) in _PIECES:
        pltpu.sync_copy(rows_s.at[pl.ds(0, nr)],
                        den_sh.at[pl.ds(sid * NPS + s0, nr)])
    plsc.subcore_barrier()

    def chunk(k, _):
        off = base + k * C
        pltpu.sync_copy(src_hbm.at[pl.ds(off, C)], sidx)
        pltpu.sync_copy(dst_hbm.at[pl.ds(off, C)], didx)
        d1 = pltpu.async_copy(acat_hbm.at[sidx], rows_s, sem)
        d2 = pltpu.async_copy(acat_hbm.at[didx], rows_d, sem)
        d1.wait()
        d2.wait()
        cv = cbuf[...]

        def grp(e, _):
            ev = rows_s[e, pl.ds(0, 16)] + rows_d[e, pl.ds(8, 16)]
            ev = jnp.maximum(ev, 0.2 * ev) - cv
            w = jnp.exp(ev)
            wbuf[e, :] = w
            rows_s[e, pl.ds(0, 16)] = w
            return 0
        lax.fori_loop(0, C, grp, 0)
        pltpu.sync_copy(wbuf, w_hbm.at[pl.ds(off, C)])
        pltpu.async_copy(rows_s, den_sh.at[didx], sem, add=True).wait()
        return 0
    lax.fori_loop(0, K, chunk, 0)

    plsc.subcore_barrier()
    for (s0, nr) in _PIECES:
        r0 = sid * NPS + s0
        pltpu.sync_copy(den_sh.at[pl.ds(r0, nr)], rows_s.at[pl.ds(0, nr)])
        pltpu.sync_copy(rows_s.at[pl.ds(0, nr)],
                        den_hbm.at[cid, pl.ds(r0, nr)])


def _edge_weights(src, dst, acat, cvec):
    fn = pl.kernel(
        _edge_weights_kernel,
        out_type=(
            jax.ShapeDtypeStruct((E_PAD, 16), F32),
            jax.ShapeDtypeStruct((2, N_PAD, 128), F32),
        ),
        mesh=_MESH,
        scratch_types=[
            pltpu.VMEM((C,), I32),
            pltpu.VMEM((C,), I32),
            pltpu.VMEM((C, 128), F32),
            pltpu.VMEM((C, 128), F32),
            pltpu.VMEM((C, 16), F32),
            pltpu.VMEM((16,), F32),
            pltpu.VMEM_SHARED((N_PAD, 128), F32),
            pltpu.SemaphoreType.DMA,
        ],
    )
    return fn(src, dst, acat, cvec)


def _agg1_kernel(src_hbm, dst_hbm, w_hbm, h_hbm, out_hbm,
                 sidx, didx, sidx2, wrows, rows, zrow, acc, sem):
    cid = lax.axis_index("c")
    sid = lax.axis_index("s")
    base = sid * 2 * KC
    lane = lax.iota(I32, 16)

    def zb(i, _):
        for r in range(HID // 16):
            zrow[i, pl.ds(r * 16, 16)] = jnp.zeros((16,), F32)
        return 0
    lax.fori_loop(0, C, zb, 0)

    for hl in range(4):
        hd = cid * 4 + hl
        for (s0, nr) in _PIECES:
            pltpu.sync_copy(zrow.at[pl.ds(0, nr)],
                            acc.at[pl.ds(sid * NPS + s0, nr)])
        plsc.subcore_barrier()

        def chunk(k, _):
            off = base + k * C
            pltpu.sync_copy(src_hbm.at[pl.ds(off, C)], sidx)
            pltpu.sync_copy(dst_hbm.at[pl.ds(off, C)], didx)
            pltpu.sync_copy(w_hbm.at[pl.ds(off, C)], wrows)

            def addoff(j, _):
                sidx2[pl.ds(j * 16, 16)] = (sidx[pl.ds(j * 16, 16)]
                                            + hd * N_PAD)
                return 0
            lax.fori_loop(0, C // 16, addoff, 0)
            pltpu.async_copy(h_hbm.at[sidx2], rows, sem).wait()
            hvec = jnp.full((16, 1), hd, I32)

            def grp(e, _):
                wrow = wrows[e, :]
                ws = _lane_bcast(wrow, hvec)
                for r in range(HID // 16):
                    sl = pl.ds(r * 16, 16)
                    rows[e, sl] = rows[e, sl] * ws
                return 0
            lax.fori_loop(0, C, grp, 0)
            pltpu.async_copy(rows, acc.at[didx], sem, add=True).wait()
            return 0
        lax.fori_loop(0, 2 * K, chunk, 0)
        plsc.subcore_barrier()
        for (s0, nr) in _PIECES:
            r0 = sid * NPS + s0
            pltpu.sync_copy(acc.at[pl.ds(r0, nr)], rows.at[pl.ds(0, nr)])
            pltpu.sync_copy(rows.at[pl.ds(0, nr)],
                            out_hbm.at[hd, pl.ds(r0, nr)])
        plsc.subcore_barrier()


def _agg1(src, dst, w1, h1flat):
    fn = pl.kernel(
        _agg1_kernel,
        out_type=jax.ShapeDtypeStruct((HEADS, N_PAD, HID), F32),
        mesh=_MESH,
        scratch_types=[
            pltpu.VMEM((C,), I32),
            pltpu.VMEM((C,), I32),
            pltpu.VMEM((C,), I32),
            pltpu.VMEM((C, 16), F32),
            pltpu.VMEM((C, HID), F32),
            pltpu.VMEM((C, HID), F32),
            pltpu.VMEM_SHARED((N_PAD, HID), F32),
            pltpu.SemaphoreType.DMA,
        ],
    )
    return fn(src, dst, w1, h1flat)


def _agg2_kernel(src_hbm, dst_hbm, w_hbm, h_hbm, out_hbm,
                 sidx, didx, sidx2, wrows, rows, zrow, acc, sem):
    cid = lax.axis_index("c")
    sid = lax.axis_index("s")
    base = sid * 2 * KC

    def zb(i, _):
        for r in range(8):
            zrow[i, pl.ds(r * 16, 16)] = jnp.zeros((16,), F32)
        return 0
    lax.fori_loop(0, C, zb, 0)

    for pp in range(2):
        q = cid * 2 + pp
        for (s0, nr) in _PIECES:
            pltpu.sync_copy(zrow.at[pl.ds(0, nr)],
                            acc.at[pl.ds(sid * NPS + s0, nr)])
        plsc.subcore_barrier()

        def chunk(k, _):
            off = base + k * C
            pltpu.sync_copy(src_hbm.at[pl.ds(off, C)], sidx)
            pltpu.sync_copy(dst_hbm.at[pl.ds(off, C)], didx)
            pltpu.sync_copy(w_hbm.at[pl.ds(off, C)], wrows)

            def addoff(j, _):
                sidx2[pl.ds(j * 16, 16)] = (sidx[pl.ds(j * 16, 16)]
                                            + q * N_PAD)
                return 0
            lax.fori_loop(0, C // 16, addoff, 0)
            pltpu.async_copy(h_hbm.at[sidx2], rows, sem).wait()

            def grp(e, _):
                wrow = wrows[e, :]
                for hh in range(2):
                    hvec = jnp.full((16, 1), q * 2 + hh, I32)
                    ws = _lane_bcast(wrow, hvec)
                    for r in range(NCLS_P // 16):
                        sl = pl.ds(hh * NCLS_P + r * 16, 16)
                        rows[e, sl] = rows[e, sl] * ws
                return 0
            lax.fori_loop(0, C, grp, 0)
            pltpu.async_copy(rows, acc.at[didx], sem, add=True).wait()
            return 0
        lax.fori_loop(0, 2 * K, chunk, 0)
        plsc.subcore_barrier()
        for (s0, nr) in _PIECES:
            r0 = sid * NPS + s0
            pltpu.sync_copy(acc.at[pl.ds(r0, nr)], rows.at[pl.ds(0, nr)])
            pltpu.sync_copy(rows.at[pl.ds(0, nr)],
                            out_hbm.at[q, pl.ds(r0, nr)])
        plsc.subcore_barrier()


def _agg2(src, dst, w2, h2flat):
    fn = pl.kernel(
        _agg2_kernel,
        out_type=jax.ShapeDtypeStruct((4, N_PAD, 128), F32),
        mesh=_MESH,
        scratch_types=[
            pltpu.VMEM((C,), I32),
            pltpu.VMEM((C,), I32),
            pltpu.VMEM((C,), I32),
            pltpu.VMEM((C, 16), F32),
            pltpu.VMEM((C, 128), F32),
            pltpu.VMEM((C, 128), F32),
            pltpu.VMEM_SHARED((N_PAD, 128), F32),
            pltpu.SemaphoreType.DMA,
        ],
    )
    return fn(src, dst, w2, h2flat)


def kernel(x, edge_index, bn_gamma, bn_beta, W1, att_src1, att_dst1, b1,
           W2, att_src2, att_dst2, b2):
    x_p = jnp.pad(x, ((0, N_PAD - N), (0, 0)))
    loop = jnp.arange(N, dtype=I32)
    npad = E_PAD - E1
    pidx = N + (jnp.arange(npad, dtype=I32) % 128)
    src = jnp.concatenate([edge_index[0].astype(I32), loop, pidx])
    dst = jnp.concatenate([edge_index[1].astype(I32), loop, pidx])
    gamma = bn_gamma.reshape(1, IN_CH)
    beta = bn_beta.reshape(1, IN_CH)
    b1r = b1.reshape(HEADS, HID)
    W2p = jnp.pad(W2.reshape(HEADS * HID, HEADS, NCLS),
                  ((0, 0), (0, 0), (0, NCLS_P - NCLS)))
    W2q = jnp.pad(W2p.reshape(HEADS * HID, 4, 2 * NCLS_P).transpose(1, 0, 2),
                  ((0, 0), (0, 0), (0, 128 - 2 * NCLS_P)))
    ats2q = jnp.pad(att_src2, ((0, 0), (0, NCLS_P - NCLS))).reshape(4, 2 * NCLS_P)
    atd2q = jnp.pad(att_dst2, ((0, 0), (0, NCLS_P - NCLS))).reshape(4, 2 * NCLS_P)
    b2p = jnp.pad(b2.reshape(HEADS, NCLS),
                  ((0, 0), (0, NCLS_P - NCLS))).reshape(1, HEADS * NCLS_P)

    stats = _bn_stats(x)
    h1, acat1 = _layer1_dense(x_p, stats, gamma, beta, W1,
                              att_src1, att_dst1)
    c1 = _cvec(acat1)[0, :16]
    w1, den1f = _edge_weights(src, dst, acat1, c1)
    den1 = den1f[:, :, :16]
    out1 = _agg1(src, dst, w1, h1.reshape(HEADS * N_PAD, HID))

    h2, acat2 = _layer2_dense(out1, den1, b1r, W2q, ats2q, atd2q)
    c2 = _cvec(acat2)[0, :16]
    w2, den2f = _edge_weights(src, dst, acat2, c2)
    den2 = den2f[:, :, :16]
    out2 = _agg2(src, dst, w2, h2.reshape(4 * N_PAD, 128))

    fin = _finalize(out2, den2, b2p)
    return fin.reshape(N_PAD, HEADS, NCLS_P)[:N, :, :NCLS].reshape(
        N, HEADS * NCLS)
